# Initial kernel scaffold; baseline (speedup 1.0000x reference)
import jax
import jax.numpy as jnp
from jax.experimental import pallas as pl


def kernel(sku, category, price, sku_table, cat_table, price_table):
    raise NotImplementedError("write your pallas kernel here")


# speedup vs baseline: 2.8746x; 2.8746x over previous
import functools

import jax
import jax.numpy as jnp
from jax import lax
from jax.experimental import pallas as pl
from jax.experimental.pallas import tpu as pltpu
from jax.experimental.pallas import tpu_sc as plsc

NC = 2
NS = 16
NW = NC * NS

CH = 128


def _build(N, D1, D2, D3, n_chunks):
    DT = D1 + D2 + D3
    mesh = plsc.VectorSubcoreMesh(core_axis_name="c", subcore_axis_name="s")

    @functools.partial(
        pl.kernel,
        out_type=jax.ShapeDtypeStruct((N, DT), jnp.float32),
        mesh=mesh,
        compiler_params=pltpu.CompilerParams(use_tc_tiling_on_sc=False),
        scratch_types=[
            pltpu.VMEM((n_chunks, CH), jnp.int32),
            pltpu.VMEM((n_chunks, CH), jnp.int32),
            pltpu.VMEM((n_chunks, CH), jnp.int32),
            pltpu.VMEM((CH, D1), jnp.float32),
            pltpu.VMEM((CH, D2), jnp.float32),
            pltpu.VMEM((CH, D3), jnp.float32),
            pltpu.SemaphoreType.DMA,
        ],
    )
    def k(idx1_hbm, idx2_hbm, idx3_hbm, t1_hbm, t2_hbm, t3_hbm, out_hbm,
          idx1_v, idx2_v, idx3_v, rows1_v, rows2_v, rows3_v, sem):
        wid = lax.axis_index("s") * NC + lax.axis_index("c")
        base = wid * (n_chunks * CH)
        pltpu.sync_copy(idx1_hbm.at[wid], idx1_v)
        pltpu.sync_copy(idx2_hbm.at[wid], idx2_v)
        pltpu.sync_copy(idx3_hbm.at[wid], idx3_v)

        def chunk(j, carry):
            pltpu.async_copy(t1_hbm.at[idx1_v.at[j]], rows1_v, sem).wait()
            pltpu.async_copy(t2_hbm.at[idx2_v.at[j]], rows2_v, sem).wait()
            pltpu.async_copy(t3_hbm.at[idx3_v.at[j]], rows3_v, sem).wait()
            r0 = base + j * CH
            pltpu.sync_copy(rows1_v, out_hbm.at[pl.ds(r0, CH), pl.ds(0, D1)])
            pltpu.sync_copy(rows2_v, out_hbm.at[pl.ds(r0, CH), pl.ds(D1, D2)])
            pltpu.sync_copy(rows3_v,
                            out_hbm.at[pl.ds(r0, CH), pl.ds(D1 + D2, D3)])
            return carry

        lax.fori_loop(0, n_chunks, chunk, 0)

    return k


def kernel(sku, category, price, sku_table, cat_table, price_table):
    Bb, Ll = sku.shape
    N = Bb * Ll
    D1 = sku_table.shape[1]
    D2 = cat_table.shape[1]
    D3 = price_table.shape[1]
    n_chunks = N // (NW * CH)
    assert n_chunks * NW * CH == N

    idx1 = sku.reshape(NW, n_chunks, CH)
    idx2 = category.reshape(NW, n_chunks, CH)
    idx3 = price.reshape(NW, n_chunks, CH)
    k = _build(N, D1, D2, D3, n_chunks)
    out = k(idx1, idx2, idx3, sku_table, cat_table, price_table)
    return out.reshape(Bb, Ll, D1 + D2 + D3)


# speedup vs baseline: 3.0072x; 1.0461x over previous
import functools

import jax
import jax.numpy as jnp
from jax import lax
from jax.experimental import pallas as pl
from jax.experimental.pallas import tpu as pltpu
from jax.experimental.pallas import tpu_sc as plsc

NC = 2
NS = 16
NW = NC * NS

CH = 128


def _build(N, D1, D2, D3, n_chunks):
    DT = D1 + D2 + D3
    assert n_chunks % 2 == 0
    mesh = plsc.VectorSubcoreMesh(core_axis_name="c", subcore_axis_name="s")

    @functools.partial(
        pl.kernel,
        out_type=jax.ShapeDtypeStruct((N, DT), jnp.float32),
        mesh=mesh,
        compiler_params=pltpu.CompilerParams(use_tc_tiling_on_sc=False),
        scratch_types=[
            pltpu.VMEM((n_chunks, CH), jnp.int32),
            pltpu.VMEM((n_chunks, CH), jnp.int32),
            pltpu.VMEM((n_chunks, CH), jnp.int32),
            pltpu.VMEM((CH, D1), jnp.float32),
            pltpu.VMEM((CH, D2), jnp.float32),
            pltpu.VMEM((CH, D3), jnp.float32),
            pltpu.VMEM((CH, D1), jnp.float32),
            pltpu.VMEM((CH, D2), jnp.float32),
            pltpu.VMEM((CH, D3), jnp.float32),
            pltpu.SemaphoreType.DMA,
            pltpu.SemaphoreType.DMA,
            pltpu.SemaphoreType.DMA,
            pltpu.SemaphoreType.DMA,
        ],
    )
    def k(idx1_hbm, idx2_hbm, idx3_hbm, t1_hbm, t2_hbm, t3_hbm, out_hbm,
          idx1_v, idx2_v, idx3_v,
          r1a, r2a, r3a, r1b, r2b, r3b, gsem0, gsem1, wsem0, wsem1):
        rows = ((r1a, r2a, r3a), (r1b, r2b, r3b))
        gsem = (gsem0, gsem1)
        wsem = (wsem0, wsem1)
        wid = lax.axis_index("s") * NC + lax.axis_index("c")
        base = wid * (n_chunks * CH)
        pltpu.sync_copy(idx1_hbm.at[wid], idx1_v)
        pltpu.sync_copy(idx2_hbm.at[wid], idx2_v)
        pltpu.sync_copy(idx3_hbm.at[wid], idx3_v)

        def g_copies(j, b):
            return (
                pltpu.make_async_copy(t1_hbm.at[idx1_v.at[j]], rows[b][0], gsem[b]),
                pltpu.make_async_copy(t2_hbm.at[idx2_v.at[j]], rows[b][1], gsem[b]),
                pltpu.make_async_copy(t3_hbm.at[idx3_v.at[j]], rows[b][2], gsem[b]),
            )

        def w_copies(j, b):
            r0 = base + j * CH
            return (
                pltpu.make_async_copy(
                    rows[b][0], out_hbm.at[pl.ds(r0, CH), pl.ds(0, D1)], wsem[b]),
                pltpu.make_async_copy(
                    rows[b][1], out_hbm.at[pl.ds(r0, CH), pl.ds(D1, D2)], wsem[b]),
                pltpu.make_async_copy(
                    rows[b][2], out_hbm.at[pl.ds(r0, CH), pl.ds(D1 + D2, D3)], wsem[b]),
            )

        def fire(cps):
            for c in cps:
                c.start()

        def drain(cps):
            for c in cps:
                c.wait()

        fire(g_copies(0, 0))

        def grp(jj, carry):
            for b in (0, 1):
                j = jj * 2 + b
                drain(g_copies(j, b))
                fire(w_copies(j, b))
                nb = 1 - b

                @pl.when(j > 0)
                def _():
                    drain(w_copies(j - 1, nb))

                @pl.when(j + 1 < n_chunks)
                def _():
                    fire(g_copies(j + 1, nb))
            return carry

        lax.fori_loop(0, n_chunks // 2, grp, 0)
        drain(w_copies(n_chunks - 1, (n_chunks - 1) % 2))

    return k


def kernel(sku, category, price, sku_table, cat_table, price_table):
    Bb, Ll = sku.shape
    N = Bb * Ll
    D1 = sku_table.shape[1]
    D2 = cat_table.shape[1]
    D3 = price_table.shape[1]
    n_chunks = N // (NW * CH)
    assert n_chunks * NW * CH == N

    idx1 = sku.reshape(NW, n_chunks, CH)
    idx2 = category.reshape(NW, n_chunks, CH)
    idx3 = price.reshape(NW, n_chunks, CH)
    k = _build(N, D1, D2, D3, n_chunks)
    out = k(idx1, idx2, idx3, sku_table, cat_table, price_table)
    return out.reshape(Bb, Ll, D1 + D2 + D3)


# speedup vs baseline: 3.0144x; 1.0024x over previous
import functools

import jax
import jax.numpy as jnp
from jax import lax
from jax.experimental import pallas as pl
from jax.experimental.pallas import tpu as pltpu
from jax.experimental.pallas import tpu_sc as plsc

NC = 2
NS = 16
NW = NC * NS

G1 = 128


def _build(Bb, Ll, D1, D2, D3):
    DT = D1 + D2 + D3
    rows_w = Bb // NW
    G2 = Ll - G1
    mesh = plsc.VectorSubcoreMesh(core_axis_name="c", subcore_axis_name="s")

    @functools.partial(
        pl.kernel,
        out_type=jax.ShapeDtypeStruct((Bb, Ll, DT), jnp.float32),
        mesh=mesh,
        compiler_params=pltpu.CompilerParams(use_tc_tiling_on_sc=False),
        scratch_types=[
            pltpu.VMEM((rows_w, Ll), jnp.int32),
            pltpu.VMEM((rows_w, Ll), jnp.int32),
            pltpu.VMEM((rows_w, Ll), jnp.int32),
            pltpu.VMEM((Ll, D1), jnp.float32),
            pltpu.VMEM((Ll, D2), jnp.float32),
            pltpu.VMEM((Ll, D3), jnp.float32),
            pltpu.VMEM((Ll, D1), jnp.float32),
            pltpu.VMEM((Ll, D2), jnp.float32),
            pltpu.VMEM((Ll, D3), jnp.float32),
            pltpu.SemaphoreType.DMA,
            pltpu.SemaphoreType.DMA,
            pltpu.SemaphoreType.DMA,
            pltpu.SemaphoreType.DMA,
        ],
    )
    def k(idx1_hbm, idx2_hbm, idx3_hbm, t1_hbm, t2_hbm, t3_hbm, out_hbm,
          idx1_v, idx2_v, idx3_v,
          r1a, r2a, r3a, r1b, r2b, r3b, gsem0, gsem1, wsem0, wsem1):
        rows = ((r1a, r2a, r3a), (r1b, r2b, r3b))
        gsem = (gsem0, gsem1)
        wsem = (wsem0, wsem1)
        wid = lax.axis_index("s") * NC + lax.axis_index("c")
        base = wid * rows_w
        pltpu.sync_copy(idx1_hbm.at[pl.ds(base, rows_w)], idx1_v)
        pltpu.sync_copy(idx2_hbm.at[pl.ds(base, rows_w)], idx2_v)
        pltpu.sync_copy(idx3_hbm.at[pl.ds(base, rows_w)], idx3_v)

        def g_copies(i, b):
            cps = []
            for t_hbm, idx_v, r in ((t1_hbm, idx1_v, rows[b][0]),
                                    (t2_hbm, idx2_v, rows[b][1]),
                                    (t3_hbm, idx3_v, rows[b][2])):
                cps.append(pltpu.make_async_copy(
                    t_hbm.at[idx_v.at[i, pl.ds(0, G1)]],
                    r.at[pl.ds(0, G1)], gsem[b]))
                cps.append(pltpu.make_async_copy(
                    t_hbm.at[idx_v.at[i, pl.ds(G1, G2)]],
                    r.at[pl.ds(G1, G2)], gsem[b]))
            return cps

        def w_copies(i, b):
            grow = base + i
            return (
                pltpu.make_async_copy(
                    rows[b][0], out_hbm.at[grow, :, pl.ds(0, D1)], wsem[b]),
                pltpu.make_async_copy(
                    rows[b][1], out_hbm.at[grow, :, pl.ds(D1, D2)], wsem[b]),
                pltpu.make_async_copy(
                    rows[b][2], out_hbm.at[grow, :, pl.ds(D1 + D2, D3)], wsem[b]),
            )

        def fire(cps):
            for c in cps:
                c.start()

        def drain(cps):
            for c in cps:
                c.wait()

        fire(g_copies(0, 0))

        def grp(jj, carry):
            for b in (0, 1):
                i = jj * 2 + b
                drain(g_copies(i, b))
                fire(w_copies(i, b))
                nb = 1 - b

                @pl.when(i > 0)
                def _():
                    drain(w_copies(i - 1, nb))

                @pl.when(i + 1 < rows_w)
                def _():
                    fire(g_copies(i + 1, nb))
            return carry

        lax.fori_loop(0, rows_w // 2, grp, 0)
        drain(w_copies(rows_w - 1, (rows_w - 1) % 2))

    return k


def kernel(sku, category, price, sku_table, cat_table, price_table):
    Bb, Ll = sku.shape
    D1 = sku_table.shape[1]
    D2 = cat_table.shape[1]
    D3 = price_table.shape[1]
    assert Bb % NW == 0 and Ll > G1
    k = _build(Bb, Ll, D1, D2, D3)
    return k(sku, category, price, sku_table, cat_table, price_table)
